# baseline (speedup 1.0000x reference)
import functools

import jax
import jax.numpy as jnp
from jax import lax
from jax.experimental import pallas as pl
from jax.experimental.pallas import tpu as pltpu
from jax.experimental.pallas import tpu_sc as plsc

_BATCH = 16384
_NF = 26
_DIM = 32
_B = _BATCH * _NF
_V = 1000000
_NC = 2
_NS = 16
_CPC = _DIM // _NC
_BPT = _B // _NS
_BB = _BATCH // _NS
_NFH = _NF // 2
_BPH = _NFH * _BB
_SEG = 62528
_SEG_LAST = _V - 15 * _SEG

_mesh = plsc.VectorSubcoreMesh(core_axis_name="c", subcore_axis_name="s")


@functools.partial(
    pl.kernel,
    mesh=_mesh,
    out_type=jax.ShapeDtypeStruct((_NF, 4, 128, 8, 128), jnp.float32),
    scratch_types=[
        pltpu.VMEM((_BPT,), jnp.int32),
        pltpu.VMEM((_BPH,), jnp.float32),
        pltpu.VMEM((_NFH * 8, 128), jnp.float32),
        pltpu.VMEM_SHARED((_V,), jnp.float32),
        pltpu.SemaphoreType.DMA,
        pltpu.SemaphoreType.DMA,
        pltpu.SemaphoreType.DMA,
    ],
    compiler_params=pltpu.CompilerParams(use_tc_tiling_on_sc=False),
)
def _sc_gather(
    idx_hbm, table_hbm, out_hbm, idx_v, vg, vg2, col_sh, gsem, wsem, ssem
):
    c = lax.axis_index("c")
    s = lax.axis_index("s")

    def idx_body(f, carry):
        pltpu.sync_copy(
            idx_hbm.at[pl.ds(f * _BATCH + s * _BB, _BB)],
            idx_v.at[pl.ds(f * _BB, _BB)],
        )
        return carry

    lax.fori_loop(0, _NF, idx_body, 0)

    def col_body(j, carry):
        d = c * _CPC + j
        g = d // 8
        r = d % 8

        plsc.subcore_barrier()

        @pl.when(s < _NS - 1)
        def _():
            pltpu.sync_copy(
                table_hbm.at[d, pl.ds(s * _SEG, _SEG)],
                col_sh.at[pl.ds(s * _SEG, _SEG)],
            )

        @pl.when(s == _NS - 1)
        def _():
            pltpu.sync_copy(
                table_hbm.at[d, pl.ds(15 * _SEG, _SEG_LAST)],
                col_sh.at[pl.ds(15 * _SEG, _SEG_LAST)],
            )

        plsc.subcore_barrier()

        for h in range(2):
            pltpu.async_copy(
                col_sh.at[idx_v.at[pl.ds(h * _BPH, _BPH)]], vg, gsem
            ).wait()

            def cp_body(i, carry):
                for u in range(8):
                    vg2[i, pl.ds(u * 16, 16)] = vg[pl.ds(i * 128 + u * 16, 16)]
                return carry

            lax.fori_loop(0, _NFH * 8, cp_body, 0)

            def w_body(ff, carry):
                pltpu.async_copy(
                    vg2.at[pl.ds(ff * 8, 8), :],
                    out_hbm.at[h * _NFH + ff, g, pl.ds(s * 8, 8), r, :],
                    wsem,
                )
                return carry

            lax.fori_loop(0, _NFH, w_body, 0)

            def drain_body(ff, carry):
                pltpu.make_async_copy(
                    vg2.at[pl.ds(ff * 8, 8), :],
                    out_hbm.at[h * _NFH + ff, g, pl.ds(s * 8, 8), r, :],
                    wsem,
                ).wait()
                return carry

            lax.fori_loop(0, _NFH, drain_body, 0)

        return carry

    lax.fori_loop(0, _CPC, col_body, 0)


def kernel(token_ids, embed_matrix):
    idx_flat = token_ids.T.reshape(-1).astype(jnp.int32)
    table_t = embed_matrix.T
    out5 = _sc_gather(idx_flat, table_t)
    return jnp.transpose(out5, (2, 4, 0, 1, 3)).reshape(_BATCH, _NF, _DIM)


# speedup vs baseline: 3.9867x; 3.9867x over previous
import functools

import jax
import jax.numpy as jnp
from jax import lax
from jax.experimental import pallas as pl
from jax.experimental.pallas import tpu as pltpu
from jax.experimental.pallas import tpu_sc as plsc

_BATCH = 16384
_NF = 26
_DIM = 32
_B = _BATCH * _NF
_V = 1000000
_NS = 32
_PER = _B // _NS
_CH = 1024
_NCH = _PER // _CH

_VB = 16384
_TG = (_V + _VB - 1) // _VB

_mesh = plsc.VectorSubcoreMesh(core_axis_name="c", subcore_axis_name="s")


def _tc_transpose_body(x_ref, o_ref):
    y = jnp.transpose(x_ref[...])
    y3 = y.reshape(_VB // 4, 4, _DIM)
    o_ref[...] = jnp.concatenate([y3[:, k, :] for k in range(4)], axis=1)


@jax.jit
def _table_to_rowmajor(table_t):
    return pl.pallas_call(
        _tc_transpose_body,
        grid=(_TG,),
        in_specs=[pl.BlockSpec((_DIM, _VB), lambda j: (0, j))],
        out_specs=pl.BlockSpec((_VB // 4, 4 * _DIM), lambda j: (j, 0)),
        out_shape=jax.ShapeDtypeStruct((_V * _DIM // 128, 128), jnp.float32),
    )(table_t)


@functools.partial(
    pl.kernel,
    mesh=_mesh,
    out_type=jax.ShapeDtypeStruct((_B, _DIM), jnp.float32),
    scratch_types=[
        pltpu.VMEM((_CH,), jnp.int32),
        pltpu.VMEM((_CH, _DIM), jnp.float32),
        pltpu.SemaphoreType.DMA,
    ],
    compiler_params=pltpu.CompilerParams(use_tc_tiling_on_sc=False),
)
def _sc_gather(idx_hbm, table_hbm, out_hbm, idx_v, rows_v, gsem):
    c = lax.axis_index("c")
    s = lax.axis_index("s")
    base = (c * (_NS // 2) + s) * _PER

    def body(i, carry):
        off = base + i * _CH
        pltpu.sync_copy(idx_hbm.at[pl.ds(off, _CH)], idx_v)
        pltpu.async_copy(table_hbm.at[idx_v], rows_v, gsem).wait()
        pltpu.sync_copy(rows_v, out_hbm.at[pl.ds(off, _CH), :])
        return carry

    lax.fori_loop(0, _NCH, body, 0)


def kernel(token_ids, embed_matrix):
    idx_flat = token_ids.reshape(-1).astype(jnp.int32)
    t_lin = _table_to_rowmajor(embed_matrix.T)
    out = _sc_gather(idx_flat, t_lin.reshape(_V, _DIM))
    return out.reshape(_BATCH, _NF, _DIM)


# speedup vs baseline: 4.6066x; 1.1555x over previous
import functools

import jax
import jax.numpy as jnp
from jax import lax
from jax.experimental import pallas as pl
from jax.experimental.pallas import tpu as pltpu
from jax.experimental.pallas import tpu_sc as plsc

_BATCH = 16384
_NF = 26
_DIM = 32
_B = _BATCH * _NF
_V = 1000000
_NS = 32
_PER = _B // _NS
_CH = 1024
_NCH = _PER // _CH

_VB = 16384
_TG = (_V + _VB - 1) // _VB

_mesh = plsc.VectorSubcoreMesh(core_axis_name="c", subcore_axis_name="s")


def _tc_transpose_body(x_ref, o_ref):
    x = x_ref[...]
    xx = jnp.concatenate([x, x, x, x], axis=0)
    t = jnp.transpose(xx)
    t4 = t.reshape(_VB // 4, 4, 4 * _DIM)
    o_ref[...] = jnp.concatenate(
        [t4[:, k, _DIM * k:_DIM * (k + 1)] for k in range(4)], axis=1
    )


@jax.jit
def _table_to_rowmajor(table_t):
    return pl.pallas_call(
        _tc_transpose_body,
        grid=(_TG,),
        in_specs=[pl.BlockSpec((_DIM, _VB), lambda j: (0, j))],
        out_specs=pl.BlockSpec((_VB // 4, 4 * _DIM), lambda j: (j, 0)),
        out_shape=jax.ShapeDtypeStruct((_V * _DIM // 128, 128), jnp.float32),
    )(table_t)


@functools.partial(
    pl.kernel,
    mesh=_mesh,
    out_type=jax.ShapeDtypeStruct((_B, _DIM), jnp.float32),
    scratch_types=[
        pltpu.VMEM((_CH,), jnp.int32),
        pltpu.VMEM((_CH, _DIM), jnp.float32),
        pltpu.SemaphoreType.DMA,
    ],
    compiler_params=pltpu.CompilerParams(use_tc_tiling_on_sc=False),
)
def _sc_gather(idx_hbm, table_hbm, out_hbm, idx_v, rows_v, gsem):
    c = lax.axis_index("c")
    s = lax.axis_index("s")
    base = (c * (_NS // 2) + s) * _PER

    def body(i, carry):
        off = base + i * _CH
        pltpu.sync_copy(idx_hbm.at[pl.ds(off, _CH)], idx_v)
        pltpu.async_copy(table_hbm.at[idx_v], rows_v, gsem).wait()
        pltpu.sync_copy(rows_v, out_hbm.at[pl.ds(off, _CH), :])
        return carry

    lax.fori_loop(0, _NCH, body, 0)


def kernel(token_ids, embed_matrix):
    idx_flat = token_ids.reshape(-1).astype(jnp.int32)
    t_lin = _table_to_rowmajor(embed_matrix.T)
    out = _sc_gather(idx_flat, t_lin.reshape(_V, _DIM))
    return out.reshape(_BATCH, _NF, _DIM)
